# Initial kernel scaffold; baseline (speedup 1.0000x reference)
import jax
import jax.numpy as jnp
from jax.experimental import pallas as pl


def kernel(atomic_numbers, bond_bases_ag, atom_graph, directed2undirected, atom_owners, atom_emb, bond_emb_W, bond_emb_b, bond_w_W, bond_w_b, conv_Wc, conv_bc, conv_Wg, conv_bg, ln_g, ln_b, mlp_W, mlp_b, out_W, out_b):
    raise NotImplementedError("write your pallas kernel here")


# speedup vs baseline: 2.8025x; 2.8025x over previous
import jax
import jax.numpy as jnp
from jax import lax
from jax.experimental import pallas as pl
from jax.experimental.pallas import tpu as pltpu
from jax.experimental.pallas import tpu_sc as plsc

NC = 2
NS = 16
NW = NC * NS
CH = 80


def _pick_nbuf(n_ch):
  for nb in (5, 4, 3, 2, 1):
    if n_ch % nb == 0:
      return nb
  return 1


def _sc_gather(table, idxs):
  t_rows, d = table.shape
  m = idxs[0].shape[0]
  per_tile = m // NW
  assert per_tile * NW == m and per_tile % CH == 0
  n_ch = per_tile // CH
  nb = _pick_nbuf(n_ch)
  n_outer = n_ch // nb
  na = len(idxs)
  mesh = plsc.VectorSubcoreMesh(core_axis_name="c", subcore_axis_name="s")
  out_type = tuple(jax.ShapeDtypeStruct((m, d), jnp.float32) for _ in range(na))
  scratch = ([pltpu.VMEM((per_tile,), jnp.int32) for _ in range(na)]
             + [pltpu.VMEM((CH, d), jnp.float32) for _ in range(na * nb)]
             + [pltpu.SemaphoreType.DMA, pltpu.SemaphoreType.DMA])

  def body(*refs):
    table_h = refs[0]
    idx_h = refs[1:1 + na]
    out_h = refs[1 + na:1 + 2 * na]
    idx_v = refs[1 + 2 * na:1 + 3 * na]
    rb = refs[1 + 3 * na:1 + 3 * na + na * nb]
    gsem, ssem = refs[-2], refs[-1]
    wid = lax.axis_index("s") * NC + lax.axis_index("c")
    base = wid * per_tile
    for a in range(na):
      pltpu.sync_copy(idx_h[a].at[pl.ds(base, per_tile)], idx_v[a])

    def outer(o, carry):
      off = o * (nb * CH)
      descs = []
      for b in range(nb):
        for a in range(na):
          dsc = pltpu.make_async_copy(
              table_h.at[idx_v[a].at[pl.ds(off + b * CH, CH)]],
              rb[a * nb + b], gsem)
          dsc.start()
          descs.append(dsc)
      for dsc in descs:
        dsc.wait()
      descs = []
      for b in range(nb):
        for a in range(na):
          dsc = pltpu.make_async_copy(
              rb[a * nb + b],
              out_h[a].at[pl.ds(base + off + b * CH, CH)], ssem)
          dsc.start()
          descs.append(dsc)
      for dsc in descs:
        dsc.wait()
      return carry

    lax.fori_loop(0, n_outer, outer, 0)

  f = pl.kernel(body, out_type=out_type, mesh=mesh, scratch_types=scratch,
                compiler_params=pltpu.CompilerParams(use_tc_tiling_on_sc=False))
  return list(f(table, *idxs))


def _sc_scatter(msg, cidx, zeros_nd):
  e, d = msg.shape
  n = zeros_nd.shape[0]
  per_tile = e // NW
  assert per_tile * NW == e and per_tile % CH == 0
  n_ch = per_tile // CH
  nb = _pick_nbuf(n_ch)
  n_outer = n_ch // nb
  rows_per_tile = n // NS
  mesh = plsc.VectorSubcoreMesh(core_axis_name="c", subcore_axis_name="s")
  out_type = jax.ShapeDtypeStruct((NC, n, d), jnp.float32)
  scratch = ([pltpu.VMEM_SHARED((n, d), jnp.float32)]
             + [pltpu.VMEM((CH,), jnp.int32) for _ in range(nb)]
             + [pltpu.VMEM((CH, d), jnp.float32) for _ in range(nb)]
             + [pltpu.SemaphoreType.DMA, pltpu.SemaphoreType.DMA])

  def body(msg_h, cidx_h, z_h, p_h, acc, *rest):
    idxb = rest[0:nb]
    msgb = rest[nb:2 * nb]
    lsem, asem = rest[-2], rest[-1]
    c = lax.axis_index("c")
    s = lax.axis_index("s")
    wid = s * NC + c
    r0 = s * rows_per_tile
    pltpu.sync_copy(z_h.at[pl.ds(r0, rows_per_tile)],
                    acc.at[pl.ds(r0, rows_per_tile)])
    plsc.subcore_barrier()
    base = wid * per_tile

    def outer(o, carry):
      off = base + o * (nb * CH)
      descs = []
      for b in range(nb):
        d1 = pltpu.make_async_copy(cidx_h.at[pl.ds(off + b * CH, CH)],
                                   idxb[b], lsem)
        d1.start()
        descs.append(d1)
        d2 = pltpu.make_async_copy(msg_h.at[pl.ds(off + b * CH, CH)],
                                   msgb[b], lsem)
        d2.start()
        descs.append(d2)
      for dsc in descs:
        dsc.wait()
      descs = []
      for b in range(nb):
        dsc = pltpu.make_async_copy(msgb[b], acc.at[idxb[b]], asem)
        dsc.start(add=True)
        descs.append(dsc)
      for dsc in descs:
        dsc.wait()
      return carry

    lax.fori_loop(0, n_outer, outer, 0)
    plsc.subcore_barrier()
    pltpu.sync_copy(acc.at[pl.ds(r0, rows_per_tile)],
                    p_h.at[c].at[pl.ds(r0, rows_per_tile)])

  f = pl.kernel(body, out_type=out_type, mesh=mesh, scratch_types=scratch,
                compiler_params=pltpu.CompilerParams(use_tc_tiling_on_sc=False))
  return f(msg, cidx, zeros_nd)


def _bond_prep_kernel(bb_ref, w1_ref, b1_ref, w2_ref, b2_ref, bf_ref, bw_ref):
  bb = bb_ref[...]
  bf_ref[...] = bb @ w1_ref[...] + b1_ref[...]
  bw_ref[...] = bb @ w2_ref[...] + b2_ref[...]


def _edge_kernel(gc_ref, gn_ref, bfd_ref, bw_ref, wc_ref, bc_ref,
                 wg_ref, bg_ref, msg_ref):
  gc = gc_ref[...]
  gn = gn_ref[...]
  bfd = bfd_ref[...]
  d = gc.shape[1]
  wc = wc_ref[...]
  wg = wg_ref[...]
  uc = (gc @ wc[:d] + gn @ wc[d:2 * d] + bfd @ wc[2 * d:]) + bc_ref[...]
  ug = (gc @ wg[:d] + gn @ wg[d:2 * d] + bfd @ wg[2 * d:]) + bg_ref[...]
  core = uc * jax.nn.sigmoid(uc)
  gate = jax.nn.sigmoid(ug)
  msg_ref[...] = core * gate * bw_ref[...]


def _combine_kernel(c_ref, p0_ref, p1_ref, o_ref):
  o_ref[...] = c_ref[...] + p0_ref[...] + p1_ref[...]


def _readout_kernel(c_ref, p0_ref, p1_ref, own_ref, lng_ref, lnb_ref,
                    mlpw_ref, mlpb_ref, outw_ref, outb_ref, out_ref,
                    sums_ref, cnts_ref):
  i = pl.program_id(0)
  nblk = pl.num_programs(0)

  @pl.when(i == 0)
  def _():
    sums_ref[...] = jnp.zeros_like(sums_ref)
    cnts_ref[...] = jnp.zeros_like(cnts_ref)

  x = c_ref[...] + p0_ref[...] + p1_ref[...]
  mu = jnp.mean(x, axis=1, keepdims=True)
  xc = x - mu
  var = jnp.mean(xc * xc, axis=1, keepdims=True)
  xn = xc * lax.rsqrt(var + 1e-5) * lng_ref[...] + lnb_ref[...]
  own = own_ref[0]
  ng = sums_ref.shape[0]
  iota = lax.broadcasted_iota(jnp.int32, (ng, own.shape[1]), 0)
  onehot = (iota == own).astype(jnp.float32)
  sums_ref[...] += lax.dot_general(onehot, xn, (((1,), (0,)), ((), ())),
                                   preferred_element_type=jnp.float32)
  cnts_ref[...] += lax.dot_general(onehot, jnp.ones_like(xn),
                                   (((1,), (0,)), ((), ())),
                                   preferred_element_type=jnp.float32)

  @pl.when(i == nblk - 1)
  def _():
    h = sums_ref[...] / jnp.maximum(cnts_ref[...], 1.0)
    for j in range(mlpw_ref.shape[0]):
      u = h @ mlpw_ref[j] + mlpb_ref[j]
      h = u * jax.nn.sigmoid(u)
    out_ref[...] = h @ outw_ref[...] + outb_ref[...]


def kernel(atomic_numbers, bond_bases_ag, atom_graph, directed2undirected,
           atom_owners, atom_emb, bond_emb_W, bond_emb_b, bond_w_W, bond_w_b,
           conv_Wc, conv_bc, conv_Wg, conv_bg, ln_g, ln_b,
           mlp_W, mlp_b, out_W, out_b):
  n = atomic_numbers.shape[0]
  e = bond_bases_ag.shape[0]
  d = atom_emb.shape[1]
  ng = 128
  na = out_W.shape[1]
  n_conv = conv_Wc.shape[0]
  n_mlp = mlp_W.shape[0]

  ba = 400
  nblk_a = n // ba
  be = 2000
  nblk_e = e // be

  m_pad = ((n + NW * CH - 1) // (NW * CH)) * (NW * CH)
  an_pad = jnp.concatenate(
      [atomic_numbers.astype(jnp.int32),
       jnp.zeros((m_pad - n,), jnp.int32)])
  a0 = _sc_gather(atom_emb, [an_pad])[0][:n]

  bf, bw = pl.pallas_call(
      _bond_prep_kernel,
      grid=(nblk_e,),
      in_specs=[
          pl.BlockSpec((be, bond_bases_ag.shape[1]), lambda i: (i, 0)),
          pl.BlockSpec(bond_emb_W.shape, lambda i: (0, 0)),
          pl.BlockSpec((1, d), lambda i: (0, 0)),
          pl.BlockSpec(bond_w_W.shape, lambda i: (0, 0)),
          pl.BlockSpec((1, d), lambda i: (0, 0)),
      ],
      out_specs=[pl.BlockSpec((be, d), lambda i: (i, 0)),
                 pl.BlockSpec((be, d), lambda i: (i, 0))],
      out_shape=[jax.ShapeDtypeStruct((e, d), jnp.float32),
                 jax.ShapeDtypeStruct((e, d), jnp.float32)],
  )(bond_bases_ag, bond_emb_W, bond_emb_b.reshape(1, d),
    bond_w_W, bond_w_b.reshape(1, d))

  bf_d = _sc_gather(bf, [directed2undirected.astype(jnp.int32)])[0]

  center = atom_graph[:, 0].astype(jnp.int32)
  nbr = atom_graph[:, 1].astype(jnp.int32)
  zeros_nd = jnp.zeros((n, d), jnp.float32)

  edge_call = pl.pallas_call(
      _edge_kernel,
      grid=(nblk_e,),
      in_specs=[
          pl.BlockSpec((be, d), lambda i: (i, 0)),
          pl.BlockSpec((be, d), lambda i: (i, 0)),
          pl.BlockSpec((be, d), lambda i: (i, 0)),
          pl.BlockSpec((be, d), lambda i: (i, 0)),
          pl.BlockSpec((3 * d, d), lambda i: (0, 0)),
          pl.BlockSpec((1, d), lambda i: (0, 0)),
          pl.BlockSpec((3 * d, d), lambda i: (0, 0)),
          pl.BlockSpec((1, d), lambda i: (0, 0)),
      ],
      out_specs=pl.BlockSpec((be, d), lambda i: (i, 0)),
      out_shape=jax.ShapeDtypeStruct((e, d), jnp.float32),
  )

  combine_call = pl.pallas_call(
      _combine_kernel,
      grid=(nblk_a,),
      in_specs=[pl.BlockSpec((ba, d), lambda i: (i, 0))] * 3,
      out_specs=pl.BlockSpec((ba, d), lambda i: (i, 0)),
      out_shape=jax.ShapeDtypeStruct((n, d), jnp.float32),
  )

  c_feas = a0
  p = None
  for i in range(n_conv):
    gc, gn = _sc_gather(c_feas, [center, nbr])
    msg = edge_call(gc, gn, bf_d, bw, conv_Wc[i], conv_bc[i].reshape(1, d),
                    conv_Wg[i], conv_bg[i].reshape(1, d))
    p = _sc_scatter(msg, center, zeros_nd)
    if i < n_conv - 1:
      c_feas = combine_call(c_feas, p[0], p[1])

  owners3 = atom_owners.astype(jnp.int32).reshape(nblk_a, 1, ba)
  out = pl.pallas_call(
      _readout_kernel,
      grid=(nblk_a,),
      in_specs=[
          pl.BlockSpec((ba, d), lambda i: (i, 0)),
          pl.BlockSpec((ba, d), lambda i: (i, 0)),
          pl.BlockSpec((ba, d), lambda i: (i, 0)),
          pl.BlockSpec((1, 1, ba), lambda i: (i, 0, 0)),
          pl.BlockSpec((1, d), lambda i: (0, 0)),
          pl.BlockSpec((1, d), lambda i: (0, 0)),
          pl.BlockSpec((n_mlp, d, d), lambda i: (0, 0, 0)),
          pl.BlockSpec((n_mlp, 1, d), lambda i: (0, 0, 0)),
          pl.BlockSpec((d, na), lambda i: (0, 0)),
          pl.BlockSpec((1, na), lambda i: (0, 0)),
      ],
      out_specs=pl.BlockSpec((ng, na), lambda i: (0, 0)),
      out_shape=jax.ShapeDtypeStruct((ng, na), jnp.float32),
      scratch_shapes=[pltpu.VMEM((ng, d), jnp.float32),
                      pltpu.VMEM((ng, d), jnp.float32)],
  )(c_feas, p[0], p[1], owners3, ln_g.reshape(1, d), ln_b.reshape(1, d),
    mlp_W, mlp_b.reshape(n_mlp, 1, d), out_W, out_b.reshape(1, na))
  return out
